# baseline (speedup 1.0000x reference)
import functools

import jax
import jax.numpy as jnp
from jax import lax
from jax.experimental import pallas as pl
from jax.experimental.pallas import tpu as pltpu
from jax.experimental.pallas import tpu_sc as plsc

N = 10000
E = 320000
D = 128
HD = 128
L = 2

NC = 2
NS = 16
NW = NC * NS
CH = 128
NCHUNK = 80
EPW = NCHUNK * CH
E_PAD = NW * EPW
NBUF = 2
NGRP = NCHUNK // NBUF
NACC = 10008
RPT = 632
R15 = 15 * RPT
RL = N - R15

_mesh = plsc.VectorSubcoreMesh(core_axis_name="c", subcore_axis_name="s")


@functools.partial(
    pl.kernel,
    out_type=jax.ShapeDtypeStruct((NC, NACC, HD), jnp.float32),
    mesh=_mesh,
    scratch_types=(
        [pltpu.VMEM((2, CH), jnp.int32)] * NBUF
        + [pltpu.VMEM((CH, HD), jnp.float32)] * NBUF
        + [pltpu.SemaphoreType.DMA] * NBUF
        + [pltpu.VMEM_SHARED((NACC, HD), jnp.float32)]
    ),
)
def _prop_kernel(s_hbm, ei_hbm, zeros_hbm, out_hbm,
                 ib0, ib1, rb0, rb1, g0, g1, acc):
    ib = (ib0, ib1)
    rows = (rb0, rb1)
    gsem = (g0, g1)
    cid = lax.axis_index("c")
    sid = lax.axis_index("s")
    wid = cid * NS + sid
    r0 = sid * RPT

    @pl.when((cid == 0) & (sid < 15))
    def _():
        pltpu.sync_copy(s_hbm.at[pl.ds(r0, RPT)], acc.at[pl.ds(r0, RPT)])

    @pl.when((cid == 0) & (sid == 15))
    def _():
        pltpu.sync_copy(s_hbm.at[pl.ds(R15, RL)], acc.at[pl.ds(R15, RL)])

    @pl.when((cid != 0) & (sid < 15))
    def _():
        pltpu.sync_copy(zeros_hbm.at[pl.ds(r0, RPT)], acc.at[pl.ds(r0, RPT)])

    @pl.when((cid != 0) & (sid == 15))
    def _():
        pltpu.sync_copy(zeros_hbm.at[pl.ds(R15, RL)], acc.at[pl.ds(R15, RL)])

    plsc.subcore_barrier()

    def load_pair(j, b):
        pltpu.sync_copy(ei_hbm.at[wid, j], ib[b])
        pltpu.async_copy(s_hbm.at[ib[b].at[0]], rows[b], gsem[b])

    for b in range(NBUF):
        load_pair(b, b)

    def body(g, carry):
        for b in range(NBUF):
            j = g * NBUF + b
            pltpu.make_async_copy(s_hbm.at[ib[b].at[0]], rows[b], gsem[b]).wait()
            pltpu.sync_copy(rows[b], acc.at[ib[b].at[1]], add=True)
            jn = j + NBUF

            @pl.when(jn < NCHUNK)
            def _(b=b, jn=jn):
                load_pair(jn, b)

        return carry

    lax.fori_loop(0, NGRP, body, 0)
    plsc.subcore_barrier()

    @pl.when(sid < 15)
    def _():
        pltpu.sync_copy(acc.at[pl.ds(r0, RPT)], out_hbm.at[cid, pl.ds(r0, RPT)])

    @pl.when(sid == 15)
    def _():
        pltpu.sync_copy(acc.at[pl.ds(R15, RL)], out_hbm.at[cid, pl.ds(R15, RL)])


R = 1000
_GRID = N // R


def _dinv_of(degp):
    deg = degp[0, :, 0:1] + degp[1, :, 0:1]
    return lax.rsqrt(deg)


def _stage_a_body(inp_ref, h_ref, degp_ref, wz_ref, wr_ref, sz_ref, sr_ref):
    dinv = _dinv_of(degp_ref[...])
    xi = inp_ref[...]
    hi = h_ref[...]
    cz = (jnp.dot(xi, wz_ref[0], preferred_element_type=jnp.float32)
          + jnp.dot(hi, wz_ref[1], preferred_element_type=jnp.float32))
    cr = (jnp.dot(xi, wr_ref[0], preferred_element_type=jnp.float32)
          + jnp.dot(hi, wr_ref[1], preferred_element_type=jnp.float32))
    sz_ref[...] = dinv * cz
    sr_ref[...] = dinv * cr


def _stage_b_body(gz_ref, gr_ref, degp_ref, inp_ref, h_ref, wh_ref,
                  bz_ref, br_ref, sh_ref, z_ref):
    dinv = _dinv_of(degp_ref[...])
    z = jax.nn.sigmoid(dinv * (gz_ref[0] + gz_ref[1]) + bz_ref[...])
    r = jax.nn.sigmoid(dinv * (gr_ref[0] + gr_ref[1]) + br_ref[...])
    ch = (jnp.dot(inp_ref[...], wh_ref[0], preferred_element_type=jnp.float32)
          + jnp.dot(r * h_ref[...], wh_ref[1], preferred_element_type=jnp.float32))
    sh_ref[...] = dinv * ch
    z_ref[...] = z


def _stage_c_body(gh_ref, degp_ref, z_ref, h_ref, bh_ref, out_ref):
    dinv = _dinv_of(degp_ref[...])
    h_tilde = jnp.tanh(dinv * (gh_ref[0] + gh_ref[1]) + bh_ref[...])
    z = z_ref[...]
    out_ref[...] = z * h_ref[...] + (1.0 - z) * h_tilde


_row = pl.BlockSpec((R, HD), lambda i: (i, 0))
_gp = pl.BlockSpec((NC, R, HD), lambda i: (0, i, 0))
_wt = pl.BlockSpec((NC, D, HD), lambda i: (0, 0, 0))
_bs = pl.BlockSpec((1, HD), lambda i: (0, 0))
_o2 = jax.ShapeDtypeStruct((N, HD), jnp.float32)

_stage_a = pl.pallas_call(
    _stage_a_body, grid=(_GRID,),
    in_specs=[_row, _row, _gp, _wt, _wt],
    out_specs=[_row, _row], out_shape=[_o2, _o2])

_stage_b = pl.pallas_call(
    _stage_b_body, grid=(_GRID,),
    in_specs=[_gp, _gp, _gp, _row, _row, _wt, _bs, _bs],
    out_specs=[_row, _row], out_shape=[_o2, _o2])

_stage_c = pl.pallas_call(
    _stage_c_body, grid=(_GRID,),
    in_specs=[_gp, _gp, _row, _row, _bs],
    out_specs=_row, out_shape=_o2)


def kernel(x, h, edge_index, Wxz, Whz, Wxr, Whr, Wxh, Whh,
           bxz, bhz, bxr, bhr, bxh, bhh):
    npad = E_PAD - E
    srcp = jnp.concatenate(
        [edge_index[0], jnp.zeros((npad,), jnp.int32)])
    dstp = jnp.concatenate(
        [edge_index[1],
         N + (jnp.arange(npad, dtype=jnp.int32) % (NACC - N))])
    ei4 = jnp.stack([srcp.reshape(NW, NCHUNK, CH),
                     dstp.reshape(NW, NCHUNK, CH)], axis=2)
    zeros128 = jnp.zeros((N, HD), jnp.float32)
    ones_col = zeros128.at[:, 0].set(1.0)

    degp = _prop_kernel(ones_col, ei4, zeros128)

    inp = x
    outs = []
    for i in range(L):
        wz = jnp.stack([Wxz[i], Whz[i]])
        wr = jnp.stack([Wxr[i], Whr[i]])
        wh = jnp.stack([Wxh[i], Whh[i]])
        bz = (bxz[i] + bhz[i])[None, :]
        br = (bxr[i] + bhr[i])[None, :]
        bh = (bxh[i] + bhh[i])[None, :]
        sz, sr = _stage_a(inp, h[i], degp, wz, wr)
        gz = _prop_kernel(sz, ei4, zeros128)
        gr = _prop_kernel(sr, ei4, zeros128)
        sh, z = _stage_b(gz, gr, degp, inp, h[i], wh, bz, br)
        gh = _prop_kernel(sh, ei4, zeros128)
        out = _stage_c(gh, degp, z, h[i], bh)
        outs.append(out)
        inp = out
    return jnp.stack(outs, axis=0)


# speedup vs baseline: 3.2525x; 3.2525x over previous
import functools

import jax
import jax.numpy as jnp
from jax import lax
from jax.experimental import pallas as pl
from jax.experimental.pallas import tpu as pltpu
from jax.experimental.pallas import tpu_sc as plsc

N = 10000
E = 320000
D = 128
HD = 128
L = 2

NC = 2
NS = 16
NW = NC * NS
EPW = E // NW
CH = 80
NCHUNK = EPW // CH
NBAT = 4
NGRP = (NCHUNK - 1) // NBAT
NP = 10240
RPT = 632
R15 = 15 * RPT
RL = N - R15
RPT8 = NP // NS

_mesh = plsc.VectorSubcoreMesh(core_axis_name="c", subcore_axis_name="s")


@functools.partial(
    pl.kernel,
    out_type=jax.ShapeDtypeStruct((NC, NP, 8), jnp.float32),
    mesh=_mesh,
    scratch_types=[
        pltpu.VMEM((NCHUNK, CH), jnp.int32),
        pltpu.VMEM((CH, 8), jnp.float32),
        pltpu.VMEM_SHARED((NP, 8), jnp.float32),
    ],
)
def _deg_kernel(dst_hbm, ones_hbm, zeros_hbm, out_hbm, dst_i, ones_v, acc):
    cid = lax.axis_index("c")
    sid = lax.axis_index("s")
    wid = cid * NS + sid
    r0 = sid * RPT8
    pltpu.sync_copy(zeros_hbm.at[pl.ds(r0, RPT8)], acc.at[pl.ds(r0, RPT8)])
    pltpu.sync_copy(ones_hbm.at[pl.ds(0, CH)], ones_v)
    pltpu.sync_copy(dst_hbm.at[wid], dst_i)
    plsc.subcore_barrier()

    def body(j, carry):
        pltpu.sync_copy(ones_v, acc.at[dst_i.at[j]], add=True)
        return carry

    lax.fori_loop(0, NCHUNK, body, 0)
    plsc.subcore_barrier()
    pltpu.sync_copy(acc.at[pl.ds(r0, RPT8)], out_hbm.at[cid, pl.ds(r0, RPT8)])


@functools.partial(
    pl.kernel,
    out_type=jax.ShapeDtypeStruct((NC, N, HD), jnp.float32),
    mesh=_mesh,
    scratch_types=(
        [pltpu.VMEM((2, CH), jnp.int32)] * NBAT
        + [pltpu.VMEM((CH, HD), jnp.float32)] * NBAT
        + [pltpu.SemaphoreType.DMA] * NBAT
        + [pltpu.VMEM_SHARED((N, HD), jnp.float32)]
    ),
)
def _prop_kernel(s_hbm, ei_hbm, zeros_hbm, out_hbm,
                 ib0, ib1, ib2, ib3, rb0, rb1, rb2, rb3,
                 g0, g1, g2, g3, acc):
    ib = (ib0, ib1, ib2, ib3)
    rows = (rb0, rb1, rb2, rb3)
    gsem = (g0, g1, g2, g3)
    cid = lax.axis_index("c")
    sid = lax.axis_index("s")
    wid = cid * NS + sid
    r0 = sid * RPT

    @pl.when((cid == 0) & (sid < 15))
    def _():
        pltpu.sync_copy(s_hbm.at[pl.ds(r0, RPT)], acc.at[pl.ds(r0, RPT)])

    @pl.when((cid == 0) & (sid == 15))
    def _():
        pltpu.sync_copy(s_hbm.at[pl.ds(R15, RL)], acc.at[pl.ds(R15, RL)])

    @pl.when((cid != 0) & (sid < 15))
    def _():
        pltpu.sync_copy(zeros_hbm.at[pl.ds(r0, RPT)], acc.at[pl.ds(r0, RPT)])

    @pl.when((cid != 0) & (sid == 15))
    def _():
        pltpu.sync_copy(zeros_hbm.at[pl.ds(R15, RL)], acc.at[pl.ds(R15, RL)])

    plsc.subcore_barrier()

    def body(g, carry):
        base = g * NBAT
        descs = []
        for b in range(NBAT):
            pltpu.sync_copy(ei_hbm.at[wid, base + b], ib[b])
            descs.append(
                pltpu.async_copy(s_hbm.at[ib[b].at[0]], rows[b], gsem[b]))
        for b in range(NBAT):
            descs[b].wait()
            pltpu.sync_copy(rows[b], acc.at[ib[b].at[1]], add=True)
        return carry

    lax.fori_loop(0, NGRP, body, 0)
    pltpu.sync_copy(ei_hbm.at[wid, NCHUNK - 1], ib[0])
    pltpu.async_copy(s_hbm.at[ib[0].at[0]], rows[0], gsem[0]).wait()
    pltpu.sync_copy(rows[0], acc.at[ib[0].at[1]], add=True)
    plsc.subcore_barrier()

    @pl.when(sid < 15)
    def _():
        pltpu.sync_copy(acc.at[pl.ds(r0, RPT)], out_hbm.at[cid, pl.ds(r0, RPT)])

    @pl.when(sid == 15)
    def _():
        pltpu.sync_copy(acc.at[pl.ds(R15, RL)], out_hbm.at[cid, pl.ds(R15, RL)])


R = 1000
_GRID = N // R


def _dinv_of(degp):
    deg = degp[0, :, 0:1] + degp[1, :, 0:1] + 1.0
    return lax.rsqrt(deg)


def _stage_a_body(inp_ref, h_ref, degp_ref, wz_ref, wr_ref, sz_ref, sr_ref):
    dinv = _dinv_of(degp_ref[...])
    xi = inp_ref[...]
    hi = h_ref[...]
    cz = (jnp.dot(xi, wz_ref[0], preferred_element_type=jnp.float32)
          + jnp.dot(hi, wz_ref[1], preferred_element_type=jnp.float32))
    cr = (jnp.dot(xi, wr_ref[0], preferred_element_type=jnp.float32)
          + jnp.dot(hi, wr_ref[1], preferred_element_type=jnp.float32))
    sz_ref[...] = dinv * cz
    sr_ref[...] = dinv * cr


def _stage_b_body(gz_ref, gr_ref, degp_ref, inp_ref, h_ref, wh_ref,
                  bz_ref, br_ref, sh_ref, z_ref):
    dinv = _dinv_of(degp_ref[...])
    z = jax.nn.sigmoid(dinv * (gz_ref[0] + gz_ref[1]) + bz_ref[...])
    r = jax.nn.sigmoid(dinv * (gr_ref[0] + gr_ref[1]) + br_ref[...])
    ch = (jnp.dot(inp_ref[...], wh_ref[0], preferred_element_type=jnp.float32)
          + jnp.dot(r * h_ref[...], wh_ref[1], preferred_element_type=jnp.float32))
    sh_ref[...] = dinv * ch
    z_ref[...] = z


def _stage_c_body(gh_ref, degp_ref, z_ref, h_ref, bh_ref, out_ref):
    dinv = _dinv_of(degp_ref[...])
    h_tilde = jnp.tanh(dinv * (gh_ref[0] + gh_ref[1]) + bh_ref[...])
    z = z_ref[...]
    out_ref[...] = z * h_ref[...] + (1.0 - z) * h_tilde


_row = pl.BlockSpec((R, HD), lambda i: (i, 0))
_gp = pl.BlockSpec((NC, R, HD), lambda i: (0, i, 0))
_dg = pl.BlockSpec((NC, R, 8), lambda i: (0, i, 0))
_wt = pl.BlockSpec((NC, D, HD), lambda i: (0, 0, 0))
_bs = pl.BlockSpec((1, HD), lambda i: (0, 0))
_o2 = jax.ShapeDtypeStruct((N, HD), jnp.float32)

_stage_a = pl.pallas_call(
    _stage_a_body, grid=(_GRID,),
    in_specs=[_row, _row, _dg, _wt, _wt],
    out_specs=[_row, _row], out_shape=[_o2, _o2])

_stage_b = pl.pallas_call(
    _stage_b_body, grid=(_GRID,),
    in_specs=[_gp, _gp, _dg, _row, _row, _wt, _bs, _bs],
    out_specs=[_row, _row], out_shape=[_o2, _o2])

_stage_c = pl.pallas_call(
    _stage_c_body, grid=(_GRID,),
    in_specs=[_gp, _dg, _row, _row, _bs],
    out_specs=_row, out_shape=_o2)


def kernel(x, h, edge_index, Wxz, Whz, Wxr, Whr, Wxh, Whh,
           bxz, bhz, bxr, bhr, bxh, bhh):
    src3 = edge_index[0].reshape(NW, NCHUNK, CH)
    dst3 = edge_index[1].reshape(NW, NCHUNK, CH)
    ei4 = jnp.stack([src3, dst3], axis=2)
    zeros128 = jnp.zeros((N, HD), jnp.float32)
    zeros8 = jnp.zeros((NP, 8), jnp.float32)
    ones_ch = jnp.ones((CH, 8), jnp.float32)

    degp = _deg_kernel(dst3, ones_ch, zeros8)

    inp = x
    outs = []
    for i in range(L):
        wz = jnp.stack([Wxz[i], Whz[i]])
        wr = jnp.stack([Wxr[i], Whr[i]])
        wh = jnp.stack([Wxh[i], Whh[i]])
        bz = (bxz[i] + bhz[i])[None, :]
        br = (bxr[i] + bhr[i])[None, :]
        bh = (bxh[i] + bhh[i])[None, :]
        sz, sr = _stage_a(inp, h[i], degp, wz, wr)
        gz = _prop_kernel(sz, ei4, zeros128)
        gr = _prop_kernel(sr, ei4, zeros128)
        sh, z = _stage_b(gz, gr, degp, inp, h[i], wh, bz, br)
        gh = _prop_kernel(sh, ei4, zeros128)
        out = _stage_c(gh, degp, z, h[i], bh)
        outs.append(out)
        inp = out
    return jnp.stack(outs, axis=0)


# speedup vs baseline: 3.6074x; 1.1091x over previous
import functools

import jax
import jax.numpy as jnp
from jax import lax
from jax.experimental import pallas as pl
from jax.experimental.pallas import tpu as pltpu
from jax.experimental.pallas import tpu_sc as plsc

N = 10000
E = 320000
D = 128
HD = 128
L = 2

NC = 2
NS = 16
NW = NC * NS
EPW = E // NW
CH = 80
NCHUNK = EPW // CH
NBAT = 4
NGRP = (NCHUNK - 1) // NBAT
NP = 10240
RPT = 632
R15 = 15 * RPT
RL = N - R15
RPT8 = NP // NS

_mesh = plsc.VectorSubcoreMesh(core_axis_name="c", subcore_axis_name="s")


@functools.partial(
    pl.kernel,
    out_type=jax.ShapeDtypeStruct((NC, NP, 8), jnp.float32),
    mesh=_mesh,
    scratch_types=[
        pltpu.VMEM((NCHUNK, CH), jnp.int32),
        pltpu.VMEM((CH, 8), jnp.float32),
        pltpu.VMEM_SHARED((NP, 8), jnp.float32),
    ],
)
def _deg_kernel(dst_hbm, ones_hbm, zeros_hbm, out_hbm, dst_i, ones_v, acc):
    cid = lax.axis_index("c")
    sid = lax.axis_index("s")
    wid = cid * NS + sid
    r0 = sid * RPT8
    pltpu.sync_copy(zeros_hbm.at[pl.ds(r0, RPT8)], acc.at[pl.ds(r0, RPT8)])
    pltpu.sync_copy(ones_hbm.at[pl.ds(0, CH)], ones_v)
    pltpu.sync_copy(dst_hbm.at[wid], dst_i)
    plsc.subcore_barrier()

    def body(j, carry):
        pltpu.sync_copy(ones_v, acc.at[dst_i.at[j]], add=True)
        return carry

    lax.fori_loop(0, NCHUNK, body, 0)
    plsc.subcore_barrier()
    pltpu.sync_copy(acc.at[pl.ds(r0, RPT8)], out_hbm.at[cid, pl.ds(r0, RPT8)])


@functools.partial(
    pl.kernel,
    out_type=jax.ShapeDtypeStruct((NC, N, HD), jnp.float32),
    mesh=_mesh,
    scratch_types=(
        [pltpu.VMEM((2, CH), jnp.int32)] * NBAT
        + [pltpu.VMEM((CH, HD), jnp.float32)] * NBAT
        + [pltpu.SemaphoreType.DMA] * (2 * NBAT)
        + [pltpu.VMEM_SHARED((N, HD), jnp.float32)]
    ),
)
def _prop_kernel(s_hbm, ei_hbm, zeros_hbm, out_hbm,
                 ib0, ib1, ib2, ib3, rb0, rb1, rb2, rb3,
                 g0, g1, g2, g3, t0, t1, t2, t3, acc):
    ib = (ib0, ib1, ib2, ib3)
    rows = (rb0, rb1, rb2, rb3)
    gsem = (g0, g1, g2, g3)
    ssem = (t0, t1, t2, t3)
    cid = lax.axis_index("c")
    sid = lax.axis_index("s")
    wid = cid * NS + sid
    r0 = sid * RPT

    @pl.when((cid == 0) & (sid < 15))
    def _():
        pltpu.sync_copy(s_hbm.at[pl.ds(r0, RPT)], acc.at[pl.ds(r0, RPT)])

    @pl.when((cid == 0) & (sid == 15))
    def _():
        pltpu.sync_copy(s_hbm.at[pl.ds(R15, RL)], acc.at[pl.ds(R15, RL)])

    @pl.when((cid != 0) & (sid < 15))
    def _():
        pltpu.sync_copy(zeros_hbm.at[pl.ds(r0, RPT)], acc.at[pl.ds(r0, RPT)])

    @pl.when((cid != 0) & (sid == 15))
    def _():
        pltpu.sync_copy(zeros_hbm.at[pl.ds(R15, RL)], acc.at[pl.ds(R15, RL)])

    plsc.subcore_barrier()

    def body(g, carry):
        base = g * NBAT
        descs = []
        for b in range(NBAT):
            @pl.when(g > 0)
            def _(b=b):
                pltpu.make_async_copy(
                    rows[b], acc.at[ib[b].at[1]], ssem[b]).wait()

            pltpu.sync_copy(ei_hbm.at[wid, base + b], ib[b])
            descs.append(
                pltpu.async_copy(s_hbm.at[ib[b].at[0]], rows[b], gsem[b]))
        for b in range(NBAT):
            descs[b].wait()
            pltpu.async_copy(rows[b], acc.at[ib[b].at[1]], ssem[b], add=True)
        return carry

    lax.fori_loop(0, NGRP, body, 0)
    for b in range(NBAT):
        pltpu.make_async_copy(rows[b], acc.at[ib[b].at[1]], ssem[b]).wait()
    pltpu.sync_copy(ei_hbm.at[wid, NCHUNK - 1], ib[0])
    pltpu.async_copy(s_hbm.at[ib[0].at[0]], rows[0], gsem[0]).wait()
    pltpu.sync_copy(rows[0], acc.at[ib[0].at[1]], add=True)
    plsc.subcore_barrier()

    @pl.when(sid < 15)
    def _():
        pltpu.sync_copy(acc.at[pl.ds(r0, RPT)], out_hbm.at[cid, pl.ds(r0, RPT)])

    @pl.when(sid == 15)
    def _():
        pltpu.sync_copy(acc.at[pl.ds(R15, RL)], out_hbm.at[cid, pl.ds(R15, RL)])


R = 1000
_GRID = N // R


def _dinv_of(degp):
    deg = degp[0, :, 0:1] + degp[1, :, 0:1] + 1.0
    return lax.rsqrt(deg)


def _stage_a_body(inp_ref, h_ref, degp_ref, wz_ref, wr_ref, sz_ref, sr_ref):
    dinv = _dinv_of(degp_ref[...])
    xi = inp_ref[...]
    hi = h_ref[...]
    cz = (jnp.dot(xi, wz_ref[0], preferred_element_type=jnp.float32)
          + jnp.dot(hi, wz_ref[1], preferred_element_type=jnp.float32))
    cr = (jnp.dot(xi, wr_ref[0], preferred_element_type=jnp.float32)
          + jnp.dot(hi, wr_ref[1], preferred_element_type=jnp.float32))
    sz_ref[...] = dinv * cz
    sr_ref[...] = dinv * cr


def _stage_b_body(gz_ref, gr_ref, degp_ref, inp_ref, h_ref, wh_ref,
                  bz_ref, br_ref, sh_ref, z_ref):
    dinv = _dinv_of(degp_ref[...])
    z = jax.nn.sigmoid(dinv * (gz_ref[0] + gz_ref[1]) + bz_ref[...])
    r = jax.nn.sigmoid(dinv * (gr_ref[0] + gr_ref[1]) + br_ref[...])
    ch = (jnp.dot(inp_ref[...], wh_ref[0], preferred_element_type=jnp.float32)
          + jnp.dot(r * h_ref[...], wh_ref[1], preferred_element_type=jnp.float32))
    sh_ref[...] = dinv * ch
    z_ref[...] = z


def _stage_c_body(gh_ref, degp_ref, z_ref, h_ref, bh_ref, out_ref):
    dinv = _dinv_of(degp_ref[...])
    h_tilde = jnp.tanh(dinv * (gh_ref[0] + gh_ref[1]) + bh_ref[...])
    z = z_ref[...]
    out_ref[...] = z * h_ref[...] + (1.0 - z) * h_tilde


_row = pl.BlockSpec((R, HD), lambda i: (i, 0))
_gp = pl.BlockSpec((NC, R, HD), lambda i: (0, i, 0))
_dg = pl.BlockSpec((NC, R, 8), lambda i: (0, i, 0))
_wt = pl.BlockSpec((NC, D, HD), lambda i: (0, 0, 0))
_bs = pl.BlockSpec((1, HD), lambda i: (0, 0))
_o2 = jax.ShapeDtypeStruct((N, HD), jnp.float32)

_stage_a = pl.pallas_call(
    _stage_a_body, grid=(_GRID,),
    in_specs=[_row, _row, _dg, _wt, _wt],
    out_specs=[_row, _row], out_shape=[_o2, _o2])

_stage_b = pl.pallas_call(
    _stage_b_body, grid=(_GRID,),
    in_specs=[_gp, _gp, _dg, _row, _row, _wt, _bs, _bs],
    out_specs=[_row, _row], out_shape=[_o2, _o2])

_stage_c = pl.pallas_call(
    _stage_c_body, grid=(_GRID,),
    in_specs=[_gp, _dg, _row, _row, _bs],
    out_specs=_row, out_shape=_o2)


def kernel(x, h, edge_index, Wxz, Whz, Wxr, Whr, Wxh, Whh,
           bxz, bhz, bxr, bhr, bxh, bhh):
    src3 = edge_index[0].reshape(NW, NCHUNK, CH)
    dst3 = edge_index[1].reshape(NW, NCHUNK, CH)
    ei4 = jnp.stack([src3, dst3], axis=2)
    zeros128 = jnp.zeros((N, HD), jnp.float32)
    zeros8 = jnp.zeros((NP, 8), jnp.float32)
    ones_ch = jnp.ones((CH, 8), jnp.float32)

    degp = _deg_kernel(dst3, ones_ch, zeros8)

    inp = x
    outs = []
    for i in range(L):
        wz = jnp.stack([Wxz[i], Whz[i]])
        wr = jnp.stack([Wxr[i], Whr[i]])
        wh = jnp.stack([Wxh[i], Whh[i]])
        bz = (bxz[i] + bhz[i])[None, :]
        br = (bxr[i] + bhr[i])[None, :]
        bh = (bxh[i] + bhh[i])[None, :]
        sz, sr = _stage_a(inp, h[i], degp, wz, wr)
        gz = _prop_kernel(sz, ei4, zeros128)
        gr = _prop_kernel(sr, ei4, zeros128)
        sh, z = _stage_b(gz, gr, degp, inp, h[i], wh, bz, br)
        gh = _prop_kernel(sh, ei4, zeros128)
        out = _stage_c(gh, degp, z, h[i], bh)
        outs.append(out)
        inp = out
    return jnp.stack(outs, axis=0)
